# baseline (speedup 1.0000x reference)
import functools

import jax
import jax.numpy as jnp
from jax import lax
from jax.experimental import pallas as pl
from jax.experimental.pallas import tpu as pltpu
from jax.experimental.pallas import tpu_sc as plsc

NUM_CORES = 2
NUM_SUBCORES = 16
NUM_WORKERS = NUM_CORES * NUM_SUBCORES

CHUNK = 128
NBUF = 4
NSEL = 2


@functools.partial(jax.jit, static_argnames=("b", "d"))
def _embed_lookup(idx_flat, table_pad, *, b, d):
    b_per_w = b // NUM_WORKERS
    n_chunks = b_per_w // CHUNK
    mesh = plsc.VectorSubcoreMesh(
        core_axis_name="c", subcore_axis_name="s",
        num_cores=NUM_CORES, num_subcores=NUM_SUBCORES,
    )

    @functools.partial(
        pl.kernel,
        mesh=mesh,
        out_type=jax.ShapeDtypeStruct((b * d // 128, 128), jnp.float32),
        scratch_types=[
            pltpu.VMEM((b_per_w,), jnp.int32),
            pltpu.VMEM((NBUF, CHUNK, 2 * d), jnp.float32),
            pltpu.VMEM((NSEL, CHUNK // 2, 2 * d), jnp.float32),
            pltpu.SemaphoreType.DMA,
            pltpu.SemaphoreType.DMA,
        ],
        compiler_params=pltpu.CompilerParams(use_tc_tiling_on_sc=False),
    )
    def run(idx_hbm, table_hbm, out_hbm, idx_v, gbuf, sbuf, gsem, osem):
        wid = lax.axis_index("s") * NUM_CORES + lax.axis_index("c")
        base = wid * b_per_w
        orow0 = base * d // 128
        orows = CHUNK * d // 128
        pltpu.sync_copy(idx_hbm.at[pl.ds(base, b_per_w)], idx_v)

        def gather(j, slot):
            return pltpu.async_copy(
                table_hbm.at[idx_v.at[pl.ds(j * CHUNK, CHUNK)]],
                gbuf.at[slot], gsem)

        def drain_gather(slot):
            pltpu.make_async_copy(
                table_hbm.at[idx_v.at[pl.ds(0, CHUNK)]], gbuf.at[slot],
                gsem).wait()

        def compact(gslot, oslot):
            src = gbuf.at[gslot]
            dst = sbuf.at[oslot]

            def tok(tau, _):
                row = src.at[tau]
                drow = dst.at[lax.div(tau, 2)]
                off = lax.rem(tau, 2) * d
                for g in range(d // 16):
                    drow[pl.ds(off + g * 16, 16)] = row[pl.ds(g * 16, 16)]
                return 0

            lax.fori_loop(0, CHUNK, tok, 0, unroll=False)

        def put(j, oslot):
            return pltpu.async_copy(
                sbuf.at[oslot],
                out_hbm.at[pl.ds(orow0 + j * orows, orows)], osem)

        def drain_put():
            pltpu.make_async_copy(
                sbuf.at[0], out_hbm.at[pl.ds(orow0, orows)], osem).wait()

        for j in range(NBUF):
            gather(j, j)
        for j in range(NSEL):
            drain_gather(j % NBUF)
            compact(j % NBUF, j % NSEL)
            gather(j + NBUF, j % NBUF)
            put(j, j % NSEL)

        def body(j, _):
            gs = lax.rem(j, NBUF)
            os_ = lax.rem(j, NSEL)
            drain_gather(gs)
            drain_put()
            compact(gs, os_)
            gather(j + NBUF, gs)
            put(j, os_)
            return 0

        lax.fori_loop(NSEL, n_chunks - NBUF, body, 0, unroll=False)

        for j in range(n_chunks - NBUF, n_chunks):
            drain_gather(j % NBUF)
            drain_put()
            compact(j % NBUF, j % NSEL)
            put(j, j % NSEL)
        for _ in range(NSEL):
            drain_put()

    return run(idx_flat, table_pad)


def kernel(token_ids, weight):
    s, t = token_ids.shape
    n, d = weight.shape
    idx_flat = token_ids.reshape(s * t).astype(jnp.int32)
    table_pad = jnp.pad(weight, ((0, 0), (0, 2 * d - weight.shape[1])))
    out2 = _embed_lookup(idx_flat, table_pad, b=s * t, d=d)
    return out2.reshape(s, t, d)
